# Initial kernel scaffold; baseline (speedup 1.0000x reference)
import jax
import jax.numpy as jnp
from jax.experimental import pallas as pl


def kernel(x, W1, g1, b1, W2, g2, b2, W3, g3, b3, W4, g4, b4, W5, g5, b5, W6, g6, b6, W7, g7, b7, W8, g8, b8, W9):
    raise NotImplementedError("write your pallas kernel here")


# speedup vs baseline: 11.2170x; 11.2170x over previous
import functools

import jax
import jax.numpy as jnp
from jax import lax
from jax.experimental import pallas as pl
from jax.experimental.pallas import tpu as pltpu
from jax.experimental.pallas import tpu_sc as plsc

B = 2
N = 4096
K = 20
KPAD = 32
NUM_CLASSES = 13


def _knn_body(xrm_ref, xcm_ref, idx_ref, *, rt, n, kk, kpad):
    xt = xrm_ref[0]
    xf = xcm_ref[0]
    g = lax.dot_general(xt, xf, (((1,), (0,)), ((), ())),
                        preferred_element_type=jnp.float32)
    d = jnp.sum(xf * xf, axis=0)
    key = 2.0 * g - d[None, :]
    col = lax.broadcasted_iota(jnp.int32, (rt, n), 1)
    base = pl.program_id(0) * n
    neginf = jnp.float32(-jnp.inf)
    cols = []
    for k in range(kk):
        m = jnp.max(key, axis=1, keepdims=True)
        am = jnp.min(jnp.where(key == m, col, n), axis=1, keepdims=True)
        cols.append(am + base)
        if k < kk - 1:
            key = jnp.where(col == am, neginf, key)
    mat = jnp.concatenate(cols + [cols[-1]] * (kpad - kk), axis=1)
    idx_ref[0] = jnp.transpose(mat)


def _knn_indices(x_rm, x_cm):
    c = x_rm.shape[2]
    rt = 128
    body = functools.partial(_knn_body, rt=rt, n=N, kk=K, kpad=KPAD)
    return pl.pallas_call(
        body,
        grid=(B, N // rt),
        in_specs=[
            pl.BlockSpec((1, rt, c), lambda b, t: (b, t, 0)),
            pl.BlockSpec((1, c, N), lambda b, t: (b, 0, 0)),
        ],
        out_specs=pl.BlockSpec((1, KPAD, rt), lambda b, t: (b, 0, t)),
        out_shape=jax.ShapeDtypeStruct((B, KPAD, N), jnp.int32),
        interpret=False,
    )(x_rm, x_cm)


def _gather_rows(table, idxt, cdim):
    nb = 1024
    gsub = 128
    chunks = N // nb
    items = B * K * chunks
    nw = 32
    per = items // nw

    mesh = plsc.VectorSubcoreMesh(core_axis_name="c", subcore_axis_name="s")

    @functools.partial(
        pl.kernel,
        out_type=jax.ShapeDtypeStruct((B, K, N, cdim), jnp.float32),
        mesh=mesh,
        scratch_types=[
            pltpu.VMEM((nb,), jnp.int32),
            pltpu.VMEM((nb, cdim), jnp.float32),
            pltpu.SemaphoreType.DMA,
        ],
        compiler_params=pltpu.CompilerParams(use_tc_tiling_on_sc=False),
    )
    def gk(table_hbm, idxt_hbm, out_hbm, idxv, rows, sem):
        wid = lax.axis_index("s") * 2 + lax.axis_index("c")

        def item_body(i, carry):
            item = wid * per + i
            b = item // (K * chunks)
            r = item % (K * chunks)
            k = r // chunks
            n0 = (r % chunks) * nb
            pltpu.sync_copy(idxt_hbm.at[b, k, pl.ds(n0, nb)], idxv)
            copies = [
                pltpu.make_async_copy(
                    table_hbm.at[idxv.at[pl.ds(j * gsub, gsub)]],
                    rows.at[pl.ds(j * gsub, gsub), :],
                    sem,
                )
                for j in range(nb // gsub)
            ]
            for cp in copies:
                cp.start()
            for cp in copies:
                cp.wait()
            pltpu.sync_copy(rows, out_hbm.at[b, k, pl.ds(n0, nb), :])
            return carry

        lax.fori_loop(0, per, item_body, 0)

    return gk(table, idxt)


def _lrelu(h):
    return jnp.where(h >= 0, h, 0.2 * h)


def _edgeconv_body2(feat_ref, xc_ref, a1_ref, bc_ref, bt1_ref, a2_ref, bt2_ref,
                    out_ref, *, kk):
    xc = xc_ref[0]
    ct = jnp.dot(xc, bc_ref[...], preferred_element_type=jnp.float32) + bt1_ref[...]
    acc = None
    for k in range(kk):
        h = jnp.dot(feat_ref[0, k], a1_ref[...],
                    preferred_element_type=jnp.float32) + ct
        h = _lrelu(h)
        h = jnp.dot(h, a2_ref[...], preferred_element_type=jnp.float32) + bt2_ref[...]
        h = _lrelu(h)
        acc = h if acc is None else jnp.maximum(acc, h)
    out_ref[0] = acc


def _edgeconv_body1(feat_ref, xc_ref, a1_ref, bc_ref, bt1_ref, out_ref, *, kk):
    xc = xc_ref[0]
    ct = jnp.dot(xc, bc_ref[...], preferred_element_type=jnp.float32) + bt1_ref[...]
    acc = None
    for k in range(kk):
        h = jnp.dot(feat_ref[0, k], a1_ref[...],
                    preferred_element_type=jnp.float32) + ct
        h = _lrelu(h)
        acc = h if acc is None else jnp.maximum(acc, h)
    out_ref[0] = acc


def _edgeconv(feat, x_rm, a1, bc, bt1, a2=None, bt2=None):
    c = x_rm.shape[2]
    nt = 1024
    wspec = lambda shp: pl.BlockSpec(shp, lambda b, t: tuple(0 for _ in shp))
    in_specs = [
        pl.BlockSpec((1, K, nt, c), lambda b, t: (b, 0, t, 0)),
        pl.BlockSpec((1, nt, c), lambda b, t: (b, t, 0)),
        wspec(a1.shape),
        wspec(bc.shape),
        wspec(bt1.shape),
    ]
    args = [feat, x_rm, a1, bc, bt1]
    if a2 is not None:
        body = functools.partial(_edgeconv_body2, kk=K)
        in_specs += [wspec(a2.shape), wspec(bt2.shape)]
        args += [a2, bt2]
    else:
        body = functools.partial(_edgeconv_body1, kk=K)
    return pl.pallas_call(
        body,
        grid=(B, N // nt),
        in_specs=in_specs,
        out_specs=pl.BlockSpec((1, nt, 64), lambda b, t: (b, t, 0)),
        out_shape=jax.ShapeDtypeStruct((B, N, 64), jnp.float32),
        interpret=False,
    )(*args)


def _head_body(x1_ref, x2_ref, x3_ref, a6_ref, bt6_ref, a7h_ref, a7x_ref,
               bt7_ref, a8_ref, bt8_ref, a9_ref, out_ref, *, nt, ntiles):
    hm = None
    for t in range(ntiles):
        sl = pl.ds(t * nt, nt)
        z = jnp.concatenate([x1_ref[0, sl], x2_ref[0, sl], x3_ref[0, sl]], axis=1)
        h6 = _lrelu(jnp.dot(z, a6_ref[...], preferred_element_type=jnp.float32)
                    + bt6_ref[...])
        mt = jnp.max(h6, axis=0, keepdims=True)
        hm = mt if hm is None else jnp.maximum(hm, mt)
    c7 = jnp.dot(hm, a7h_ref[...], preferred_element_type=jnp.float32)
    for t in range(ntiles):
        sl = pl.ds(t * nt, nt)
        z = jnp.concatenate([x1_ref[0, sl], x2_ref[0, sl], x3_ref[0, sl]], axis=1)
        h7 = _lrelu(jnp.dot(z, a7x_ref[...], preferred_element_type=jnp.float32)
                    + c7 + bt7_ref[...])
        h8 = _lrelu(jnp.dot(h7, a8_ref[...], preferred_element_type=jnp.float32)
                    + bt8_ref[...])
        out_ref[0, sl] = jnp.dot(h8, a9_ref[...], preferred_element_type=jnp.float32)


def _head(x1, x2, x3, a6, bt6, a7h, a7x, bt7, a8, bt8, a9):
    nt = 1024
    body = functools.partial(_head_body, nt=nt, ntiles=N // nt)
    wspec = lambda a: pl.BlockSpec(a.shape, lambda b: tuple(0 for _ in a.shape))
    xspec = pl.BlockSpec((1, N, 64), lambda b: (b, 0, 0))
    return pl.pallas_call(
        body,
        grid=(B,),
        in_specs=[xspec, xspec, xspec] + [wspec(a) for a in
                                          (a6, bt6, a7h, a7x, bt7, a8, bt8, a9)],
        out_specs=pl.BlockSpec((1, N, 16), lambda b: (b, 0, 0)),
        out_shape=jax.ShapeDtypeStruct((B, N, 16), jnp.float32),
        interpret=False,
    )(x1, x2, x3, a6, bt6, a7h, a7x, bt7, a8, bt8, a9)


def kernel(x, W1, g1, b1, W2, g2, b2, W3, g3, b3, W4, g4, b4, W5, g5, b5,
           W6, g6, b6, W7, g7, b7, W8, g8, b8, W9):
    s = jnp.float32(1.0) / jnp.sqrt(jnp.float32(1.0 + 1e-5))

    def fold_edge(W, g, c, cpad):
        gs = (g * s)[None, :]
        wa = jnp.transpose(W[:, :c])
        wb = jnp.transpose(W[:, c:])
        a = wa * gs
        bcm = (wb - wa) * gs
        if cpad > c:
            a = jnp.pad(a, ((0, cpad - c), (0, 0)))
            bcm = jnp.pad(bcm, ((0, cpad - c), (0, 0)))
        return a, bcm

    def fold_lin(W, g):
        return jnp.transpose(W) * (g * s)[None, :]

    x_cm = jnp.pad(x, ((0, 0), (0, 7), (0, 0)))
    x_rm = jnp.transpose(x_cm, (0, 2, 1))
    idx1 = _knn_indices(x_rm, x_cm)
    f1 = _gather_rows(x_rm.reshape(B * N, 16), idx1, 16)
    a1, bc1 = fold_edge(W1, g1, 9, 16)
    a2 = fold_lin(W2, g2)
    x1 = _edgeconv(f1, x_rm, a1, bc1, b1[None, :], a2, b2[None, :])

    x1_cm = jnp.transpose(x1, (0, 2, 1))
    idx2 = _knn_indices(x1, x1_cm)
    f2 = _gather_rows(x1.reshape(B * N, 64), idx2, 64)
    a3, bc3 = fold_edge(W3, g3, 64, 64)
    a4 = fold_lin(W4, g4)
    x2 = _edgeconv(f2, x1, a3, bc3, b3[None, :], a4, b4[None, :])

    x2_cm = jnp.transpose(x2, (0, 2, 1))
    idx3 = _knn_indices(x2, x2_cm)
    f3 = _gather_rows(x2.reshape(B * N, 64), idx3, 64)
    a5, bc5 = fold_edge(W5, g5, 64, 64)
    x3 = _edgeconv(f3, x2, a5, bc5, b5[None, :])

    a6 = fold_lin(W6, g6)
    a7 = fold_lin(W7, g7)
    a8 = fold_lin(W8, g8)
    a9 = jnp.pad(jnp.transpose(W9), ((0, 0), (0, 16 - NUM_CLASSES)))
    out = _head(x1, x2, x3, a6, b6[None, :], a7[:1024], a7[1024:],
                b7[None, :], a8, b8[None, :], a9)
    return jnp.transpose(out[:, :, :NUM_CLASSES], (0, 2, 1))


# speedup vs baseline: 14.9798x; 1.3355x over previous
import functools

import jax
import jax.numpy as jnp
from jax import lax
from jax.experimental import pallas as pl
from jax.experimental.pallas import tpu as pltpu
from jax.experimental.pallas import tpu_sc as plsc

B = 2
N = 4096
K = 20
KPAD = 32
NUM_CLASSES = 13


def _knn_body(xrm_ref, xcm_ref, idx_ref, *, rt, n, kk, kpad):
    xt = xrm_ref[...]
    xf = xcm_ref[...]
    g = lax.dot_general(xt, xf, (((1,), (0,)), ((), ())),
                        preferred_element_type=jnp.float32)
    d = jnp.sum(xf * xf, axis=0)
    key = 2.0 * g - d[None, :]
    colf = lax.broadcasted_iota(jnp.int32, (rt, n), 1).astype(jnp.float32)
    neginf = jnp.float32(-jnp.inf)
    bigf = jnp.float32(n)
    cols = []
    m = jnp.max(key, axis=1, keepdims=True)
    for k in range(kk):
        hit = key == m
        am = jnp.min(jnp.where(hit, colf, bigf), axis=1, keepdims=True)
        cols.append(am.astype(jnp.int32))
        if k < kk - 1:
            key = jnp.where(hit, neginf, key)
            m = jnp.max(key, axis=1, keepdims=True)
    mat = jnp.concatenate(cols + [cols[-1]] * (kpad - kk), axis=1)
    idx_ref[...] = jnp.transpose(mat)


def _knn_indices(x_rm, x_cm):
    c = x_rm.shape[1]
    rt = 256
    body = functools.partial(_knn_body, rt=rt, n=N, kk=K, kpad=KPAD)
    return pl.pallas_call(
        body,
        grid=(N // rt,),
        in_specs=[
            pl.BlockSpec((rt, c), lambda t: (t, 0)),
            pl.BlockSpec((c, N), lambda t: (0, 0)),
        ],
        out_specs=pl.BlockSpec((KPAD, rt), lambda t: (0, t)),
        out_shape=jax.ShapeDtypeStruct((KPAD, N), jnp.int32),
        interpret=False,
    )(x_rm, x_cm)


def _gather_rows(table, idxt, cdim):
    nb = 512
    gsub = 128
    chunks = N // nb
    items = K * chunks
    nw = 32
    per = items // nw

    mesh = plsc.VectorSubcoreMesh(core_axis_name="c", subcore_axis_name="s")

    @functools.partial(
        pl.kernel,
        out_type=jax.ShapeDtypeStruct((K, N, cdim), jnp.float32),
        mesh=mesh,
        scratch_types=[
            pltpu.VMEM((nb,), jnp.int32),
            pltpu.VMEM((nb, cdim), jnp.float32),
            pltpu.SemaphoreType.DMA,
        ],
        compiler_params=pltpu.CompilerParams(use_tc_tiling_on_sc=False),
    )
    def gk(table_hbm, idxt_hbm, out_hbm, idxv, rows, sem):
        wid = lax.axis_index("s") * 2 + lax.axis_index("c")

        def item_body(i, carry):
            item = wid * per + i
            k = item // chunks
            n0 = (item % chunks) * nb
            pltpu.sync_copy(idxt_hbm.at[k, pl.ds(n0, nb)], idxv)
            copies = [
                pltpu.make_async_copy(
                    table_hbm.at[idxv.at[pl.ds(j * gsub, gsub)]],
                    rows.at[pl.ds(j * gsub, gsub), :],
                    sem,
                )
                for j in range(nb // gsub)
            ]
            for cp in copies:
                cp.start()
            for cp in copies:
                cp.wait()
            pltpu.sync_copy(rows, out_hbm.at[k, pl.ds(n0, nb), :])
            return carry

        lax.fori_loop(0, per, item_body, 0)

    return gk(table, idxt)


def _lrelu(h):
    return jnp.where(h >= 0, h, 0.2 * h)


def _edgeconv_body2(feat_ref, xc_ref, a1_ref, bc_ref, bt1_ref, a2_ref, bt2_ref,
                    out_ref, outc_ref, *, kk):
    xc = xc_ref[...]
    ct = jnp.dot(xc, bc_ref[...], preferred_element_type=jnp.float32) + bt1_ref[...]
    acc = None
    for k in range(kk):
        h = jnp.dot(feat_ref[k], a1_ref[...],
                    preferred_element_type=jnp.float32) + ct
        h = _lrelu(h)
        h = jnp.dot(h, a2_ref[...], preferred_element_type=jnp.float32) + bt2_ref[...]
        h = _lrelu(h)
        acc = h if acc is None else jnp.maximum(acc, h)
    out_ref[...] = acc
    outc_ref[...] = jnp.transpose(acc)


def _edgeconv_body1(feat_ref, xc_ref, a1_ref, bc_ref, bt1_ref, out_ref,
                    outc_ref, *, kk):
    xc = xc_ref[...]
    ct = jnp.dot(xc, bc_ref[...], preferred_element_type=jnp.float32) + bt1_ref[...]
    acc = None
    for k in range(kk):
        h = jnp.dot(feat_ref[k], a1_ref[...],
                    preferred_element_type=jnp.float32) + ct
        h = _lrelu(h)
        acc = h if acc is None else jnp.maximum(acc, h)
    out_ref[...] = acc
    outc_ref[...] = jnp.transpose(acc)


def _edgeconv(feat, x_rm, a1, bc, bt1, a2=None, bt2=None):
    c = x_rm.shape[1]
    nt = 1024
    wspec = lambda shp: pl.BlockSpec(shp, lambda t: tuple(0 for _ in shp))
    in_specs = [
        pl.BlockSpec((K, nt, c), lambda t: (0, t, 0)),
        pl.BlockSpec((nt, c), lambda t: (t, 0)),
        wspec(a1.shape),
        wspec(bc.shape),
        wspec(bt1.shape),
    ]
    args = [feat, x_rm, a1, bc, bt1]
    if a2 is not None:
        body = functools.partial(_edgeconv_body2, kk=K)
        in_specs += [wspec(a2.shape), wspec(bt2.shape)]
        args += [a2, bt2]
    else:
        body = functools.partial(_edgeconv_body1, kk=K)
    return pl.pallas_call(
        body,
        grid=(N // nt,),
        in_specs=in_specs,
        out_specs=[
            pl.BlockSpec((nt, 64), lambda t: (t, 0)),
            pl.BlockSpec((64, nt), lambda t: (0, t)),
        ],
        out_shape=[
            jax.ShapeDtypeStruct((N, 64), jnp.float32),
            jax.ShapeDtypeStruct((64, N), jnp.float32),
        ],
        interpret=False,
    )(*args)


def _head_body(x1_ref, x2_ref, x3_ref, a6_ref, bt6_ref, a7h_ref, a7x_ref,
               bt7_ref, a8_ref, bt8_ref, a9_ref, out_ref, *, nt, ntiles):
    hm = None
    for t in range(ntiles):
        sl = pl.ds(t * nt, nt)
        z = jnp.concatenate([x1_ref[sl], x2_ref[sl], x3_ref[sl]], axis=1)
        h6 = _lrelu(jnp.dot(z, a6_ref[...], preferred_element_type=jnp.float32)
                    + bt6_ref[...])
        mt = jnp.max(h6, axis=0, keepdims=True)
        hm = mt if hm is None else jnp.maximum(hm, mt)
    c7 = jnp.dot(hm, a7h_ref[...], preferred_element_type=jnp.float32)
    for t in range(ntiles):
        sl = pl.ds(t * nt, nt)
        z = jnp.concatenate([x1_ref[sl], x2_ref[sl], x3_ref[sl]], axis=1)
        h7 = _lrelu(jnp.dot(z, a7x_ref[...], preferred_element_type=jnp.float32)
                    + c7 + bt7_ref[...])
        h8 = _lrelu(jnp.dot(h7, a8_ref[...], preferred_element_type=jnp.float32)
                    + bt8_ref[...])
        out_ref[sl, :] = jnp.dot(h8, a9_ref[...], preferred_element_type=jnp.float32)


def _head(x1, x2, x3, a6, bt6, a7h, a7x, bt7, a8, bt8, a9):
    nt = 1024
    body = functools.partial(_head_body, nt=nt, ntiles=N // nt)
    wspec = lambda a: pl.BlockSpec(a.shape, lambda: tuple(0 for _ in a.shape))
    xspec = pl.BlockSpec((N, 64), lambda: (0, 0))
    return pl.pallas_call(
        body,
        grid=(),
        in_specs=[xspec, xspec, xspec] + [wspec(a) for a in
                                          (a6, bt6, a7h, a7x, bt7, a8, bt8, a9)],
        out_specs=pl.BlockSpec((N, 16), lambda: (0, 0)),
        out_shape=jax.ShapeDtypeStruct((N, 16), jnp.float32),
        interpret=False,
    )(x1, x2, x3, a6, bt6, a7h, a7x, bt7, a8, bt8, a9)


def kernel(x, W1, g1, b1, W2, g2, b2, W3, g3, b3, W4, g4, b4, W5, g5, b5,
           W6, g6, b6, W7, g7, b7, W8, g8, b8, W9):
    s = jnp.float32(1.0) / jnp.sqrt(jnp.float32(1.0 + 1e-5))

    def fold_edge(W, g, c, cpad):
        gs = (g * s)[None, :]
        wa = jnp.transpose(W[:, :c])
        wb = jnp.transpose(W[:, c:])
        a = wa * gs
        bcm = (wb - wa) * gs
        if cpad > c:
            a = jnp.pad(a, ((0, cpad - c), (0, 0)))
            bcm = jnp.pad(bcm, ((0, cpad - c), (0, 0)))
        return a, bcm

    def fold_lin(W, g):
        return jnp.transpose(W) * (g * s)[None, :]

    a1, bc1 = fold_edge(W1, g1, 9, 16)
    a2 = fold_lin(W2, g2)
    a3, bc3 = fold_edge(W3, g3, 64, 64)
    a4 = fold_lin(W4, g4)
    a5, bc5 = fold_edge(W5, g5, 64, 64)
    a6 = fold_lin(W6, g6)
    a7 = fold_lin(W7, g7)
    a8 = fold_lin(W8, g8)
    a9 = jnp.pad(jnp.transpose(W9), ((0, 0), (0, 16 - NUM_CLASSES)))

    x_cm = jnp.pad(x, ((0, 0), (0, 7), (0, 0)))
    x_rm = jnp.transpose(x_cm, (0, 2, 1))

    outs = []
    idx1 = [None] * B
    f1 = [None] * B
    x1 = [None] * B
    idx2 = [None] * B
    f2 = [None] * B
    x2 = [None] * B
    idx3 = [None] * B
    f3 = [None] * B
    x3 = [None] * B
    for b in range(B):
        idx1[b] = _knn_indices(x_rm[b], x_cm[b])
    for b in range(B):
        f1[b] = _gather_rows(x_rm[b], idx1[b], 16)
    for b in range(B):
        x1[b] = _edgeconv(f1[b], x_rm[b], a1, bc1, b1[None, :], a2, b2[None, :])
    for b in range(B):
        idx2[b] = _knn_indices(x1[b][0], x1[b][1])
    for b in range(B):
        f2[b] = _gather_rows(x1[b][0], idx2[b], 64)
    for b in range(B):
        x2[b] = _edgeconv(f2[b], x1[b][0], a3, bc3, b3[None, :], a4, b4[None, :])
    for b in range(B):
        idx3[b] = _knn_indices(x2[b][0], x2[b][1])
    for b in range(B):
        f3[b] = _gather_rows(x2[b][0], idx3[b], 64)
    for b in range(B):
        x3[b] = _edgeconv(f3[b], x2[b][0], a5, bc5, b5[None, :])
    for b in range(B):
        outs.append(_head(x1[b][0], x2[b][0], x3[b][0], a6, b6[None, :],
                          a7[:1024], a7[1024:], b7[None, :], a8, b8[None, :], a9))
    out = jnp.stack(outs)
    return jnp.transpose(out[:, :, :NUM_CLASSES], (0, 2, 1))


# speedup vs baseline: 15.0679x; 1.0059x over previous
import functools

import jax
import jax.numpy as jnp
from jax import lax
from jax.experimental import pallas as pl
from jax.experimental.pallas import tpu as pltpu
from jax.experimental.pallas import tpu_sc as plsc

B = 2
N = 4096
K = 20
KPAD = 32
NUM_CLASSES = 13


def _knn_body(xrm_ref, xcm_ref, idx_ref, *, rt, n, kk, kpad):
    xt = xrm_ref[0]
    xf = xcm_ref[0]
    g = lax.dot_general(xt, xf, (((1,), (0,)), ((), ())),
                        preferred_element_type=jnp.float32)
    d = jnp.sum(xf * xf, axis=0)
    key = 2.0 * g - d[None, :]
    colf = lax.broadcasted_iota(jnp.int32, (rt, n), 1).astype(jnp.float32)
    base = pl.program_id(0) * n
    neginf = jnp.float32(-jnp.inf)
    bigf = jnp.float32(n)
    cols = []
    m = jnp.max(key, axis=1, keepdims=True)
    for k in range(kk):
        hit = key == m
        am = jnp.min(jnp.where(hit, colf, bigf), axis=1, keepdims=True)
        cols.append(am.astype(jnp.int32) + base)
        if k < kk - 1:
            key = jnp.where(hit, neginf, key)
            m = jnp.max(key, axis=1, keepdims=True)
    mat = jnp.concatenate(cols + [cols[-1]] * (kpad - kk), axis=1)
    idx_ref[0] = jnp.transpose(mat)


def _knn_indices(x_rm, x_cm):
    c = x_rm.shape[2]
    rt = 256
    body = functools.partial(_knn_body, rt=rt, n=N, kk=K, kpad=KPAD)
    return pl.pallas_call(
        body,
        grid=(B, N // rt),
        in_specs=[
            pl.BlockSpec((1, rt, c), lambda b, t: (b, t, 0)),
            pl.BlockSpec((1, c, N), lambda b, t: (b, 0, 0)),
        ],
        out_specs=pl.BlockSpec((1, KPAD, rt), lambda b, t: (b, 0, t)),
        out_shape=jax.ShapeDtypeStruct((B, KPAD, N), jnp.int32),
        interpret=False,
    )(x_rm, x_cm)


def _gather_rows(table, idxt, cdim):
    nb = 1024
    gsub = 128
    chunks = N // nb
    items = B * K * chunks
    nw = 32
    per = items // nw

    mesh = plsc.VectorSubcoreMesh(core_axis_name="c", subcore_axis_name="s")

    @functools.partial(
        pl.kernel,
        out_type=jax.ShapeDtypeStruct((B, K, N, cdim), jnp.float32),
        mesh=mesh,
        scratch_types=[
            pltpu.VMEM((nb,), jnp.int32),
            pltpu.VMEM((nb, cdim), jnp.float32),
            pltpu.SemaphoreType.DMA,
        ],
        compiler_params=pltpu.CompilerParams(use_tc_tiling_on_sc=False),
    )
    def gk(table_hbm, idxt_hbm, out_hbm, idxv, rows, sem):
        wid = lax.axis_index("s") * 2 + lax.axis_index("c")

        def item_body(i, carry):
            item = wid * per + i
            b = item // (K * chunks)
            r = item % (K * chunks)
            k = r // chunks
            n0 = (r % chunks) * nb
            pltpu.sync_copy(idxt_hbm.at[b, k, pl.ds(n0, nb)], idxv)
            copies = [
                pltpu.make_async_copy(
                    table_hbm.at[idxv.at[pl.ds(j * gsub, gsub)]],
                    rows.at[pl.ds(j * gsub, gsub), :],
                    sem,
                )
                for j in range(nb // gsub)
            ]
            for cp in copies:
                cp.start()
            for cp in copies:
                cp.wait()
            pltpu.sync_copy(rows, out_hbm.at[b, k, pl.ds(n0, nb), :])
            return carry

        lax.fori_loop(0, per, item_body, 0)

    return gk(table, idxt)


def _lrelu(h):
    return jnp.where(h >= 0, h, 0.2 * h)


def _edgeconv_body2(feat_ref, xc_ref, a1_ref, bc_ref, bt1_ref, a2_ref, bt2_ref,
                    out_ref, outc_ref, *, kk):
    xc = xc_ref[0]
    ct = jnp.dot(xc, bc_ref[...], preferred_element_type=jnp.float32) + bt1_ref[...]
    acc = None
    for k in range(kk):
        h = jnp.dot(feat_ref[0, k], a1_ref[...],
                    preferred_element_type=jnp.float32) + ct
        h = _lrelu(h)
        h = jnp.dot(h, a2_ref[...], preferred_element_type=jnp.float32) + bt2_ref[...]
        h = _lrelu(h)
        acc = h if acc is None else jnp.maximum(acc, h)
    out_ref[0] = acc
    outc_ref[0] = jnp.transpose(acc)


def _edgeconv_body1(feat_ref, xc_ref, a1_ref, bc_ref, bt1_ref, out_ref, *, kk):
    xc = xc_ref[0]
    ct = jnp.dot(xc, bc_ref[...], preferred_element_type=jnp.float32) + bt1_ref[...]
    acc = None
    for k in range(kk):
        h = jnp.dot(feat_ref[0, k], a1_ref[...],
                    preferred_element_type=jnp.float32) + ct
        h = _lrelu(h)
        acc = h if acc is None else jnp.maximum(acc, h)
    out_ref[0] = acc


def _edgeconv(feat, x_rm, a1, bc, bt1, a2=None, bt2=None):
    c = x_rm.shape[2]
    nt = 1024
    wspec = lambda shp: pl.BlockSpec(shp, lambda b, t: tuple(0 for _ in shp))
    in_specs = [
        pl.BlockSpec((1, K, nt, c), lambda b, t: (b, 0, t, 0)),
        pl.BlockSpec((1, nt, c), lambda b, t: (b, t, 0)),
        wspec(a1.shape),
        wspec(bc.shape),
        wspec(bt1.shape),
    ]
    args = [feat, x_rm, a1, bc, bt1]
    if a2 is not None:
        body = functools.partial(_edgeconv_body2, kk=K)
        in_specs += [wspec(a2.shape), wspec(bt2.shape)]
        args += [a2, bt2]
    else:
        body = functools.partial(_edgeconv_body1, kk=K)
    out_specs = pl.BlockSpec((1, nt, 64), lambda b, t: (b, t, 0))
    out_shape = jax.ShapeDtypeStruct((B, N, 64), jnp.float32)
    if a2 is not None:
        out_specs = [out_specs, pl.BlockSpec((1, 64, nt), lambda b, t: (b, 0, t))]
        out_shape = [out_shape, jax.ShapeDtypeStruct((B, 64, N), jnp.float32)]
    return pl.pallas_call(
        body,
        grid=(B, N // nt),
        in_specs=in_specs,
        out_specs=out_specs,
        out_shape=out_shape,
        interpret=False,
    )(*args)


def _head_body(x1_ref, x2_ref, x3_ref, a6_ref, bt6_ref, a7h_ref, a7x_ref,
               bt7_ref, a8_ref, bt8_ref, a9_ref, out_ref, *, nt, ntiles):
    hm = None
    for t in range(ntiles):
        sl = pl.ds(t * nt, nt)
        z = jnp.concatenate([x1_ref[0, sl], x2_ref[0, sl], x3_ref[0, sl]], axis=1)
        h6 = _lrelu(jnp.dot(z, a6_ref[...], preferred_element_type=jnp.float32)
                    + bt6_ref[...])
        mt = jnp.max(h6, axis=0, keepdims=True)
        hm = mt if hm is None else jnp.maximum(hm, mt)
    c7 = jnp.dot(hm, a7h_ref[...], preferred_element_type=jnp.float32)
    for t in range(ntiles):
        sl = pl.ds(t * nt, nt)
        z = jnp.concatenate([x1_ref[0, sl], x2_ref[0, sl], x3_ref[0, sl]], axis=1)
        h7 = _lrelu(jnp.dot(z, a7x_ref[...], preferred_element_type=jnp.float32)
                    + c7 + bt7_ref[...])
        h8 = _lrelu(jnp.dot(h7, a8_ref[...], preferred_element_type=jnp.float32)
                    + bt8_ref[...])
        out_ref[0, sl] = jnp.dot(h8, a9_ref[...], preferred_element_type=jnp.float32)


def _head(x1, x2, x3, a6, bt6, a7h, a7x, bt7, a8, bt8, a9):
    nt = 1024
    body = functools.partial(_head_body, nt=nt, ntiles=N // nt)
    wspec = lambda a: pl.BlockSpec(a.shape, lambda b: tuple(0 for _ in a.shape))
    xspec = pl.BlockSpec((1, N, 64), lambda b: (b, 0, 0))
    return pl.pallas_call(
        body,
        grid=(B,),
        in_specs=[xspec, xspec, xspec] + [wspec(a) for a in
                                          (a6, bt6, a7h, a7x, bt7, a8, bt8, a9)],
        out_specs=pl.BlockSpec((1, N, 16), lambda b: (b, 0, 0)),
        out_shape=jax.ShapeDtypeStruct((B, N, 16), jnp.float32),
        interpret=False,
    )(x1, x2, x3, a6, bt6, a7h, a7x, bt7, a8, bt8, a9)


def kernel(x, W1, g1, b1, W2, g2, b2, W3, g3, b3, W4, g4, b4, W5, g5, b5,
           W6, g6, b6, W7, g7, b7, W8, g8, b8, W9):
    s = jnp.float32(1.0) / jnp.sqrt(jnp.float32(1.0 + 1e-5))

    def fold_edge(W, g, c, cpad):
        gs = (g * s)[None, :]
        wa = jnp.transpose(W[:, :c])
        wb = jnp.transpose(W[:, c:])
        a = wa * gs
        bcm = (wb - wa) * gs
        if cpad > c:
            a = jnp.pad(a, ((0, cpad - c), (0, 0)))
            bcm = jnp.pad(bcm, ((0, cpad - c), (0, 0)))
        return a, bcm

    def fold_lin(W, g):
        return jnp.transpose(W) * (g * s)[None, :]

    x_cm = jnp.pad(x, ((0, 0), (0, 7), (0, 0)))
    x_rm = jnp.transpose(x_cm, (0, 2, 1))
    idx1 = _knn_indices(x_rm, x_cm)
    f1 = _gather_rows(x_rm.reshape(B * N, 16), idx1, 16)
    a1, bc1 = fold_edge(W1, g1, 9, 16)
    a2 = fold_lin(W2, g2)
    x1, x1_cm = _edgeconv(f1, x_rm, a1, bc1, b1[None, :], a2, b2[None, :])

    idx2 = _knn_indices(x1, x1_cm)
    f2 = _gather_rows(x1.reshape(B * N, 64), idx2, 64)
    a3, bc3 = fold_edge(W3, g3, 64, 64)
    a4 = fold_lin(W4, g4)
    x2, x2_cm = _edgeconv(f2, x1, a3, bc3, b3[None, :], a4, b4[None, :])

    idx3 = _knn_indices(x2, x2_cm)
    f3 = _gather_rows(x2.reshape(B * N, 64), idx3, 64)
    a5, bc5 = fold_edge(W5, g5, 64, 64)
    x3 = _edgeconv(f3, x2, a5, bc5, b5[None, :])

    a6 = fold_lin(W6, g6)
    a7 = fold_lin(W7, g7)
    a8 = fold_lin(W8, g8)
    a9 = jnp.pad(jnp.transpose(W9), ((0, 0), (0, 16 - NUM_CLASSES)))
    out = _head(x1, x2, x3, a6, b6[None, :], a7[:1024], a7[1024:],
                b7[None, :], a8, b8[None, :], a9)
    return jnp.transpose(out[:, :, :NUM_CLASSES], (0, 2, 1))
